# baseline (speedup 1.0000x reference)
import functools

import jax
import jax.numpy as jnp
from jax import lax
from jax.experimental import pallas as pl
from jax.experimental.pallas import tpu as pltpu
from jax.experimental.pallas import tpu_sc as plsc

_VOCAB = 1000000
_D = 32
_BATCH = 16384
_HIST = 50
_B = _BATCH * _HIST
_NC = 2
_NS = 16
_NW = _NC * _NS
_BPW = _B // _NW
_CHUNK = 128
_NCH = _BPW // _CHUNK
_K = 10
_NG = _NCH // _K
_NGP = _NG // 2

_mesh = plsc.VectorSubcoreMesh(core_axis_name="c", subcore_axis_name="s")


@functools.partial(
    pl.kernel,
    out_type=jax.ShapeDtypeStruct((_NW, _NCH, _CHUNK, _D), jnp.float32),
    mesh=_mesh,
    scratch_types=[
        pltpu.VMEM((_NCH, _CHUNK), jnp.int32),
        pltpu.VMEM((2, _K, _CHUNK, _D), jnp.float32),
        pltpu.SemaphoreType.DMA,
        pltpu.SemaphoreType.DMA,
        pltpu.SemaphoreType.DMA,
        pltpu.SemaphoreType.DMA,
    ],
    compiler_params=pltpu.CompilerParams(use_tc_tiling_on_sc=False),
)
def _gather(ids_hbm, table_hbm, out_hbm, idx_v, rows_v,
            gsem0, gsem1, osem0, osem1):
    wid = lax.axis_index("s") * _NC + lax.axis_index("c")
    pltpu.sync_copy(ids_hbm.at[wid], idx_v)

    def fire_gathers(g, half, sem):
        for b in range(_K):
            pltpu.async_copy(table_hbm.at[idx_v.at[g * _K + b]],
                             rows_v.at[half, b], sem)

    def drain_gathers(g, half, sem):
        for b in range(_K):
            pltpu.make_async_copy(table_hbm.at[idx_v.at[g * _K + b]],
                                  rows_v.at[half, b], sem).wait()

    def fire_outs(g, half, sem):
        pltpu.async_copy(rows_v.at[half],
                         out_hbm.at[wid, pl.ds(g * _K, _K)], sem)

    def drain_outs(g, half, sem):
        pltpu.make_async_copy(rows_v.at[half],
                              out_hbm.at[wid, pl.ds(g * _K, _K)], sem).wait()

    fire_gathers(0, 0, gsem0)

    @pl.loop(0, _NGP)
    def _pair(gp):
        g0 = 2 * gp
        g1 = g0 + 1
        drain_gathers(g0, 0, gsem0)
        fire_gathers(g1, 1, gsem1)
        fire_outs(g0, 0, osem0)
        drain_gathers(g1, 1, gsem1)
        fire_outs(g1, 1, osem1)
        drain_outs(g0, 0, osem0)

        @pl.when(gp + 1 < _NGP)
        def _():
            fire_gathers(g0 + 2, 0, gsem0)

        drain_outs(g1, 1, osem1)


def kernel(ids, length, table):
    del length
    ids_r = ids.reshape(_NW, _NCH, _CHUNK)
    out = _gather(ids_r, table)
    return out.reshape(_BATCH, _HIST, _D)


# speedup vs baseline: 1.3479x; 1.3479x over previous
import functools

import jax
import jax.numpy as jnp
from jax import lax
from jax.experimental import pallas as pl
from jax.experimental.pallas import tpu as pltpu
from jax.experimental.pallas import tpu_sc as plsc

_VOCAB = 1000000
_D = 32
_BATCH = 16384
_HIST = 50
_NC = 2
_NS = 16
_NW = _NC * _NS
_RPW = _BATCH // _NW
_K = 8
_NG = _RPW // _K
_NGP = _NG // 2

_mesh = plsc.VectorSubcoreMesh(core_axis_name="c", subcore_axis_name="s")


@functools.partial(
    pl.kernel,
    out_type=jax.ShapeDtypeStruct((_BATCH, _HIST, _D), jnp.float32),
    mesh=_mesh,
    scratch_types=[
        pltpu.VMEM((_RPW, _HIST), jnp.int32),
        pltpu.VMEM((2, _K, _HIST, _D), jnp.float32),
        pltpu.SemaphoreType.DMA,
        pltpu.SemaphoreType.DMA,
        pltpu.SemaphoreType.DMA,
        pltpu.SemaphoreType.DMA,
    ],
    compiler_params=pltpu.CompilerParams(use_tc_tiling_on_sc=False),
)
def _gather(ids_hbm, table_hbm, out_hbm, idx_v, rows_v,
            gsem0, gsem1, osem0, osem1):
    wid = lax.axis_index("s") * _NC + lax.axis_index("c")
    row0 = wid * _RPW
    pltpu.sync_copy(ids_hbm.at[pl.ds(row0, _RPW)], idx_v)

    def fire_gathers(g, half, sem):
        for b in range(_K):
            pltpu.async_copy(table_hbm.at[idx_v.at[g * _K + b]],
                             rows_v.at[half, b], sem)

    def drain_gathers(g, half, sem):
        for b in range(_K):
            pltpu.make_async_copy(table_hbm.at[idx_v.at[g * _K + b]],
                                  rows_v.at[half, b], sem).wait()

    def fire_outs(g, half, sem):
        pltpu.async_copy(rows_v.at[half],
                         out_hbm.at[pl.ds(row0 + g * _K, _K)], sem)

    def drain_outs(g, half, sem):
        pltpu.make_async_copy(rows_v.at[half],
                              out_hbm.at[pl.ds(row0 + g * _K, _K)], sem).wait()

    fire_gathers(0, 0, gsem0)

    @pl.loop(0, _NGP)
    def _pair(gp):
        g0 = 2 * gp
        g1 = g0 + 1
        drain_gathers(g0, 0, gsem0)
        fire_gathers(g1, 1, gsem1)
        fire_outs(g0, 0, osem0)
        drain_gathers(g1, 1, gsem1)
        fire_outs(g1, 1, osem1)
        drain_outs(g0, 0, osem0)

        @pl.when(gp + 1 < _NGP)
        def _():
            fire_gathers(g0 + 2, 0, gsem0)

        drain_outs(g1, 1, osem1)


def kernel(ids, length, table):
    del length
    return _gather(ids, table)
